# baseline (speedup 1.0000x reference)
import functools

import jax
import jax.numpy as jnp
from jax import lax
from jax.experimental import pallas as pl
from jax.experimental.pallas import tpu as pltpu
from jax.experimental.pallas import tpu_sc as plsc

N_USERS = 1000000
N_ASPECTS = 8
BATCH = 16384
A_MIN, A_MAX = 1.0, 5.0
R_MIN, R_MAX = 1.0, 5.0

_NC = 2
_NS = 16
_NW = _NC * _NS
_BPW = BATCH // _NW
_CHUNK = 128
_NCHUNK = _BPW // _CHUNK
_GROUPS = _BPW // 16


@functools.partial(
    pl.kernel,
    mesh=plsc.VectorSubcoreMesh(core_axis_name="c", subcore_axis_name="s"),
    out_type=jax.ShapeDtypeStruct((BATCH,), jnp.float32),
    compiler_params=pltpu.CompilerParams(
        needs_layout_passes=False, use_tc_tiling_on_sc=False
    ),
    scratch_types=[
        pltpu.VMEM((_NCHUNK, _CHUNK), jnp.int32),
        pltpu.VMEM((_BPW, N_ASPECTS), jnp.float32),
        pltpu.VMEM((N_ASPECTS, _BPW), jnp.float32),
        pltpu.VMEM((_BPW,), jnp.float32),
        pltpu.SemaphoreType.DMA,
    ],
)
def _sc_predict(u_hbm, a_hbm, table_hbm, out_hbm, idx_v, rows_v, a_v, out_v, sem):
    wid = lax.axis_index("s") * _NC + lax.axis_index("c")
    base = wid * _BPW

    for c in range(_NCHUNK):
        pltpu.sync_copy(u_hbm.at[pl.ds(base + c * _CHUNK, _CHUNK)], idx_v.at[c])
    copies = []
    for c in range(_NCHUNK):
        copies.append(
            pltpu.async_copy(
                table_hbm.at[idx_v.at[c]],
                rows_v.at[pl.ds(c * _CHUNK, _CHUNK)],
                sem,
            )
        )
    for j in range(N_ASPECTS):
        pltpu.sync_copy(a_hbm.at[j, pl.ds(base, _BPW)], a_v.at[j])
    for cp in copies:
        cp.wait()

    lane = lax.iota(jnp.int32, 16)

    def group_body(g, _):
        ii = g * 16 + lane
        acc = jnp.zeros((16,), jnp.float32)
        l1 = jnp.zeros((16,), jnp.float32)
        for j in range(N_ASPECTS):
            jj = jnp.full((16,), j, jnp.int32)
            w = plsc.load_gather(rows_v, [ii, jj])
            a = a_v[j, pl.ds(g * 16, 16)]
            acc = acc + w * a
            l1 = l1 + jnp.abs(w)
        pred = acc / jnp.maximum(l1, 1e-12)
        out_v[pl.ds(g * 16, 16)] = R_MIN + (R_MIN - R_MAX) * (
            (pred - A_MIN) / (A_MAX - A_MIN)
        )
        return 0

    lax.fori_loop(0, _GROUPS, group_body, 0)
    pltpu.sync_copy(out_v, out_hbm.at[pl.ds(base, _BPW)])


def kernel(U_ids, A_ratings, users_parameters):
    return _sc_predict(U_ids.astype(jnp.int32), A_ratings, users_parameters)
